# baseline (speedup 1.0000x reference)
import functools

import jax
import jax.numpy as jnp
from jax import lax
from jax.experimental import pallas as pl
from jax.experimental.pallas import tpu as pltpu
from jax.experimental.pallas import tpu_sc as plsc

NC = 2
NS = 16
LANES = 16
NW = NC * NS


@functools.lru_cache(maxsize=None)
def _build(B, S, D):
    C = S // NW
    T = 16
    NT = C // T
    ND = D // LANES
    assert S % NW == 0 and C % T == 0 and D % LANES == 0

    mesh = plsc.VectorSubcoreMesh(
        core_axis_name="c", subcore_axis_name="s",
        num_cores=NC, num_subcores=NS)

    @functools.partial(
        pl.kernel,
        out_type=jax.ShapeDtypeStruct((B, S, D), jnp.float32),
        mesh=mesh,
        scratch_types=[
            pltpu.VMEM((T, D), jnp.float32),
            pltpu.VMEM((T, D), jnp.float32),
        ],
    )
    def k(x_hbm, tab_hbm, out_hbm, tbuf, xbuf):
        cid = lax.axis_index("c")
        sid = lax.axis_index("s")
        wid = sid * NC + cid
        base = wid * C

        def tile_body(t, carry):
            p = base + t * T
            pltpu.sync_copy(tab_hbm.at[pl.ds(p, T)], tbuf)
            for b in range(B):
                pltpu.sync_copy(x_hbm.at[b, pl.ds(p, T)], xbuf)

                def row_body(r, c2):
                    for j in range(ND):
                        sl = pl.ds(j * LANES, LANES)
                        xbuf[r, sl] = xbuf[r, sl] + tbuf[r, sl]
                    return c2

                lax.fori_loop(0, T, row_body, 0, unroll=False)
                pltpu.sync_copy(xbuf, out_hbm.at[b, pl.ds(p, T)])
            return carry

        lax.fori_loop(0, NT, tile_body, 0, unroll=False)

    return k


def kernel(x, pos_table):
    B, S, D = x.shape
    return _build(B, S, D)(x, pos_table[:S])


# speedup vs baseline: 1.4880x; 1.4880x over previous
import functools

import jax
import jax.numpy as jnp
from jax import lax
from jax.experimental import pallas as pl
from jax.experimental.pallas import tpu as pltpu
from jax.experimental.pallas import tpu_sc as plsc

NC = 2
NS = 16
LANES = 16
NW = NC * NS


@functools.lru_cache(maxsize=None)
def _build(B, S, D):
    C = S // NW
    T = 16
    NT = C // T
    ND = D // LANES
    assert S % NW == 0 and C % (2 * T) == 0 and D % LANES == 0

    mesh = plsc.VectorSubcoreMesh(
        core_axis_name="c", subcore_axis_name="s",
        num_cores=NC, num_subcores=NS)

    scratch = [
        pltpu.VMEM((T, D), jnp.float32),
        pltpu.VMEM((T, D), jnp.float32),
    ]
    scratch += [pltpu.VMEM((T, D), jnp.float32) for _ in range(B)]
    scratch += [pltpu.SemaphoreType.DMA for _ in range(2 + 2 * B)]

    @functools.partial(
        pl.kernel,
        out_type=jax.ShapeDtypeStruct((B, S, D), jnp.float32),
        mesh=mesh,
        scratch_types=scratch,
    )
    def k(x_hbm, tab_hbm, out_hbm, *bufs):
        tbufs = bufs[0:2]
        xbufs = bufs[2:2 + B]
        sem_t = bufs[2 + B:4 + B]
        sem_x = bufs[4 + B:4 + 2 * B]
        sem_s = bufs[4 + 2 * B:4 + 3 * B]

        cid = lax.axis_index("c")
        sid = lax.axis_index("s")
        wid = sid * NC + cid
        base = wid * C

        def load_tab(t, which):
            pltpu.async_copy(tab_hbm.at[pl.ds(base + t * T, T)],
                             tbufs[which], sem_t[which])

        def load_x(t, b):
            pltpu.async_copy(x_hbm.at[b, pl.ds(base + t * T, T)],
                             xbufs[b], sem_x[b])

        def wait(src, dst, sem):
            pltpu.make_async_copy(src, dst, sem).wait()

        load_tab(0, 0)
        for b in range(B):
            load_x(0, b)

        def half(t, cur):
            p = base + t * T
            more = t + 1 < NT

            @pl.when(more)
            def _():
                load_tab(t + 1, 1 - cur)

            tb = tbufs[cur]
            wait(tab_hbm.at[pl.ds(p, T)], tb, sem_t[cur])
            for b in range(B):
                xb = xbufs[b]
                wait(x_hbm.at[b, pl.ds(p, T)], xb, sem_x[b])

                def row_body(r, c2, xb=xb, tb=tb):
                    for j in range(ND):
                        sl = pl.ds(j * LANES, LANES)
                        xb[r, sl] = xb[r, sl] + tb[r, sl]
                    return c2

                lax.fori_loop(0, T, row_body, 0, unroll=False)
                pltpu.async_copy(xb, out_hbm.at[b, pl.ds(p, T)], sem_s[b])

            @pl.when(more)
            def _():
                for b in range(B):
                    wait(xbufs[b], out_hbm.at[b, pl.ds(p, T)], sem_s[b])
                    load_x(t + 1, b)

        def pair_body(i, carry):
            half(2 * i, 0)
            half(2 * i + 1, 1)
            return carry

        lax.fori_loop(0, NT // 2, pair_body, 0, unroll=False)

        p_last = base + (NT - 1) * T
        for b in range(B):
            wait(xbufs[b], out_hbm.at[b, pl.ds(p_last, T)], sem_s[b])

    return k


def kernel(x, pos_table):
    B, S, D = x.shape
    return _build(B, S, D)(x, pos_table[:S])


# speedup vs baseline: 2.0978x; 1.4099x over previous
import functools

import jax
import jax.numpy as jnp
from jax import lax
from jax.experimental import pallas as pl
from jax.experimental.pallas import tpu as pltpu
from jax.experimental.pallas import tpu_sc as plsc

NC = 2
NS = 16
LANES = 16
NW = NC * NS


@functools.lru_cache(maxsize=None)
def _build(B, S, D):
    C = S // NW
    T = 8
    NT = C // T
    ND = D // LANES
    assert S % NW == 0 and C % (2 * T) == 0 and D % LANES == 0

    mesh = plsc.VectorSubcoreMesh(
        core_axis_name="c", subcore_axis_name="s",
        num_cores=NC, num_subcores=NS)

    scratch = [pltpu.VMEM((T, D), jnp.float32) for _ in range(2)]
    scratch += [pltpu.VMEM((T, D), jnp.float32) for _ in range(2 * B)]
    scratch += [pltpu.SemaphoreType.DMA for _ in range(2 + 4 * B)]

    @functools.partial(
        pl.kernel,
        out_type=jax.ShapeDtypeStruct((B, S, D), jnp.float32),
        mesh=mesh,
        scratch_types=scratch,
    )
    def k(x_hbm, tab_hbm, out_hbm, *bufs):
        tbufs = bufs[0:2]
        xbufs = bufs[2:2 + 2 * B]
        sem_t = bufs[2 + 2 * B:4 + 2 * B]
        sem_x = bufs[4 + 2 * B:4 + 4 * B]
        sem_s = bufs[4 + 4 * B:4 + 6 * B]

        cid = lax.axis_index("c")
        sid = lax.axis_index("s")
        wid = sid * NC + cid
        base = wid * C

        def load_tab(t, q):
            pltpu.async_copy(tab_hbm.at[pl.ds(base + t * T, T)],
                             tbufs[q], sem_t[q])

        def load_x(t, b, q):
            pltpu.async_copy(x_hbm.at[b, pl.ds(base + t * T, T)],
                             xbufs[2 * b + q], sem_x[2 * b + q])

        def wait(src, dst, sem):
            pltpu.make_async_copy(src, dst, sem).wait()

        load_tab(0, 0)
        for b in range(B):
            load_x(0, b, 0)

        def half(t, q, first, last):
            p = base + t * T
            tb = tbufs[q]

            if not last:
                load_tab(t + 1, 1 - q)
            wait(tab_hbm.at[pl.ds(p, T)], tb, sem_t[q])

            for b in range(B):
                xb = xbufs[2 * b + q]
                ob = xbufs[2 * b + (1 - q)]
                wait(x_hbm.at[b, pl.ds(p, T)], xb, sem_x[2 * b + q])

                def row_body(r, c2, xb=xb, tb=tb):
                    for j in range(ND):
                        sl = pl.ds(j * LANES, LANES)
                        xb[r, sl] = xb[r, sl] + tb[r, sl]
                    return c2

                lax.fori_loop(0, T, row_body, 0, unroll=False)
                pltpu.async_copy(xb, out_hbm.at[b, pl.ds(p, T)],
                                 sem_s[2 * b + q])

                if not last:
                    def reload(b=b, ob=ob, t=t, q=q):
                        wait(ob, out_hbm.at[b, pl.ds(p - T, T)],
                             sem_s[2 * b + (1 - q)])
                        load_x(t + 1, b, 1 - q)

                    if first:
                        @pl.when(t > 0)
                        def _():
                            reload()

                        @pl.when(t == 0)
                        def _(b=b, t=t, q=q):
                            load_x(t + 1, b, 1 - q)
                    else:
                        reload()

        def pair_body(i, carry):
            half(2 * i, 0, first=True, last=False)
            half(2 * i + 1, 1, first=False, last=(NT == 2))
            return carry

        lax.fori_loop(0, NT // 2 - 1, pair_body, 0, unroll=False)
        half(NT - 2, 0, first=False, last=False)
        half(NT - 1, 1, first=False, last=True)

        for b in range(B):
            wait(xbufs[2 * b + 0], out_hbm.at[b, pl.ds(base + (NT - 2) * T, T)],
                 sem_s[2 * b + 0])
            wait(xbufs[2 * b + 1], out_hbm.at[b, pl.ds(base + (NT - 1) * T, T)],
                 sem_s[2 * b + 1])

    return k


def kernel(x, pos_table):
    B, S, D = x.shape
    return _build(B, S, D)(x, pos_table[:S])
